# baseline (speedup 1.0000x reference)
import functools

import jax
import jax.numpy as jnp
from jax import lax
from jax.experimental import pallas as pl
from jax.experimental.pallas import tpu as pltpu
from jax.experimental.pallas import tpu_sc as plsc

_NC = 2
_NS = 16
_NW = _NC * _NS
_G = 128
_LANES = 16


@functools.partial(jax.jit, static_argnames=("seq", "batch", "emb"))
def _sc_embed(table, idx2, pe2, *, seq, batch, emb):
    n_flat = seq * batch
    nchunks = n_flat // _G
    per_w = nchunks // _NW
    chunks_per_l = batch // _G

    mesh = plsc.VectorSubcoreMesh(core_axis_name="c", subcore_axis_name="s")

    @functools.partial(
        pl.kernel,
        out_type=jax.ShapeDtypeStruct((n_flat, emb), jnp.float32),
        mesh=mesh,
        scratch_types=[
            pltpu.VMEM((seq, emb), jnp.float32),
            pltpu.VMEM((1, _G), jnp.int32),
            pltpu.VMEM((_G, emb), jnp.float32),
            pltpu.SemaphoreType.DMA,
        ],
        compiler_params=pltpu.CompilerParams(use_tc_tiling_on_sc=False),
    )
    def k(table_hbm, idx_hbm, pe_hbm, out_hbm, pe_v, idx_v, rows_v, sem):
        wid = lax.axis_index("s") * _NC + lax.axis_index("c")
        pltpu.sync_copy(pe_hbm, pe_v)

        @pl.loop(0, per_w)
        def _(ci):
            gi = wid * per_w + ci
            pltpu.sync_copy(idx_hbm.at[pl.ds(gi, 1)], idx_v)
            pltpu.async_copy(table_hbm.at[idx_v.at[0]], rows_v, sem).wait()
            l = gi // chunks_per_l
            pe_regs = [
                pe_v[pl.ds(l, 1), pl.ds(c * _LANES, _LANES)]
                for c in range(emb // _LANES)
            ]

            @pl.loop(0, _G)
            def _(r):
                for c in range(emb // _LANES):
                    slc = (pl.ds(r, 1), pl.ds(c * _LANES, _LANES))
                    rows_v[slc] = rows_v[slc] + pe_regs[c]

            pltpu.sync_copy(rows_v, out_hbm.at[pl.ds(gi * _G, _G)])

    return k(table, idx2, pe2)


def kernel(input, table, pe):
    batch, seq = input.shape
    emb = table.shape[1]
    idx2 = input.T.reshape(seq * batch // _G, _G)
    pe2 = pe.reshape(pe.shape[0], emb)[:seq]
    out = _sc_embed(table, idx2, pe2, seq=seq, batch=batch, emb=emb)
    return out.reshape(seq, batch, emb)
